# Initial kernel scaffold; baseline (speedup 1.0000x reference)
import jax
import jax.numpy as jnp
from jax.experimental import pallas as pl


def kernel(adj_indices, adj_values, x, W, b):
    raise NotImplementedError("write your pallas kernel here")


# speedup vs baseline: 4.4539x; 4.4539x over previous
import functools

import jax
import jax.numpy as jnp
from jax import lax
from jax.experimental import pallas as pl
from jax.experimental.pallas import tpu as pltpu
from jax.experimental.pallas import tpu_sc as plsc

N = 10000
E = 320000
F = 128

NC = 2
NS = 16
NW = NC * NS
EPT = E // NW
K = 80
CHUNKS = EPT // K
NP = 10240
RPT = NP // NS


def _mm_body(x_ref, w_ref, o_ref):
    o_ref[...] = jnp.dot(x_ref[...], w_ref[...],
                         preferred_element_type=jnp.float32)


def _matmul(x, W):
    return pl.pallas_call(
        _mm_body,
        grid=(10,),
        in_specs=[
            pl.BlockSpec((N // 10, F), lambda i: (i, 0)),
            pl.BlockSpec((F, F), lambda i: (0, 0)),
        ],
        out_specs=pl.BlockSpec((N // 10, F), lambda i: (i, 0)),
        out_shape=jax.ShapeDtypeStruct((N, F), jnp.float32),
    )(x, W)


def _add_body(p_ref, b_ref, o_ref):
    o_ref[...] = p_ref[0] + p_ref[1] + b_ref[...]


def _final_add(partials, b):
    return pl.pallas_call(
        _add_body,
        grid=(10,),
        in_specs=[
            pl.BlockSpec((2, N // 10, F), lambda i: (0, i, 0)),
            pl.BlockSpec((1, F), lambda i: (0, 0)),
        ],
        out_specs=pl.BlockSpec((N // 10, F), lambda i: (i, 0)),
        out_shape=jax.ShapeDtypeStruct((N, F), jnp.float32),
    )(partials, b.reshape(1, F))


def _spmm_body(support_hbm, rows_hbm, cols_hbm, vals_hbm, out_hbm,
               rows_c, cols_c, vals_c, msgs, acc, sem):
    c = lax.axis_index("c")
    s = lax.axis_index("s")
    ebase = (c * NS + s) * EPT

    zv = jnp.zeros((16,), jnp.float32)

    def zero_row(j, carry):
        for v in range(F // 16):
            msgs[j, pl.ds(v * 16, 16)] = zv
        return carry

    lax.fori_loop(0, K, zero_row, 0)
    for i in range(RPT // K):
        pltpu.sync_copy(msgs, acc.at[pl.ds(s * RPT + i * K, K)])
    plsc.subcore_barrier()

    def chunk_body(g, carry):
        off = ebase + g * K
        pltpu.sync_copy(cols_hbm.at[pl.ds(off, K)], cols_c)
        pltpu.sync_copy(rows_hbm.at[pl.ds(off, K)], rows_c)
        pltpu.sync_copy(vals_hbm.at[pl.ds(off, K)], vals_c)
        pltpu.async_copy(support_hbm.at[cols_c], msgs, sem).wait()

        def group_body(t, carry2):
            base = t * 16
            vv = vals_c[pl.ds(base, 16)]
            for l in range(16):
                val = vv[l]
                for v in range(F // 16):
                    sl = pl.ds(v * 16, 16)
                    msgs[base + l, sl] = msgs[base + l, sl] * val
            return carry2

        lax.fori_loop(0, K // 16, group_body, 0)
        pltpu.sync_copy(msgs, acc.at[rows_c], add=True)
        return carry

    lax.fori_loop(0, CHUNKS, chunk_body, 0)
    plsc.subcore_barrier()

    for i in range(RPT // K):
        base = s * RPT + i * K
        pltpu.sync_copy(acc.at[pl.ds(base, K)], msgs)
        pltpu.sync_copy(msgs, out_hbm.at[c, pl.ds(base, K)])


def _spmm(support, rows, cols, vals):
    mesh = plsc.VectorSubcoreMesh(core_axis_name="c", subcore_axis_name="s",
                                  num_cores=NC, num_subcores=NS)
    f = pl.kernel(
        _spmm_body,
        out_type=jax.ShapeDtypeStruct((NC, NP, F), jnp.float32),
        mesh=mesh,
        scratch_types=[
            pltpu.VMEM((K,), jnp.int32),
            pltpu.VMEM((K,), jnp.int32),
            pltpu.VMEM((K,), jnp.float32),
            pltpu.VMEM((K, F), jnp.float32),
            pltpu.VMEM_SHARED((NP, F), jnp.float32),
            pltpu.SemaphoreType.DMA,
        ],
    )
    return f(support, rows, cols, vals)


@jax.jit
def kernel(adj_indices, adj_values, x, W, b):
    x = x.astype(jnp.float32)
    support = _matmul(x, W)
    partials = _spmm(support, adj_indices[0], adj_indices[1], adj_values)
    return _final_add(partials, b)


# speedup vs baseline: 6.8585x; 1.5399x over previous
import jax
import jax.numpy as jnp
from jax import lax
from jax.experimental import pallas as pl
from jax.experimental.pallas import tpu as pltpu
from jax.experimental.pallas import tpu_sc as plsc

N = 10000
E = 320000
F = 128

NC = 2
NS = 16
NW = NC * NS
EPT = E // NW
K = 80
CHUNKS = EPT // K
T = E // K
NP = 10240
RPT = NP // NS


def _mm_body(x_ref, w_ref, o_ref):
    o_ref[...] = jnp.dot(x_ref[...], w_ref[...],
                         preferred_element_type=jnp.float32)


def _matmul(x, W):
    return pl.pallas_call(
        _mm_body,
        grid=(10,),
        in_specs=[
            pl.BlockSpec((N // 10, F), lambda i: (i, 0)),
            pl.BlockSpec((F, F), lambda i: (0, 0)),
        ],
        out_specs=pl.BlockSpec((N // 10, F), lambda i: (i, 0)),
        out_shape=jax.ShapeDtypeStruct((N, F), jnp.float32),
    )(x, W)


def _add_body(p_ref, b_ref, o_ref):
    o_ref[...] = p_ref[0] + p_ref[1] + b_ref[...]


def _final_add(partials, b):
    return pl.pallas_call(
        _add_body,
        grid=(10,),
        in_specs=[
            pl.BlockSpec((2, N // 10, F), lambda i: (0, i, 0)),
            pl.BlockSpec((1, F), lambda i: (0, 0)),
        ],
        out_specs=pl.BlockSpec((N // 10, F), lambda i: (i, 0)),
        out_shape=jax.ShapeDtypeStruct((N, F), jnp.float32),
    )(partials, b.reshape(1, F))


def _spmm_body(support_hbm, pk_hbm, vals_hbm, out_hbm,
               idx_a, idx_b, vals_va, vals_vb, msgs_a, msgs_b, acc,
               sem_a, sem_b):
    c = lax.axis_index("c")
    s = lax.axis_index("s")
    tbase = (c * NS + s) * CHUNKS

    zv = jnp.zeros((16,), jnp.float32)

    def zero_row(j, carry):
        for v in range(F // 16):
            msgs_a[j, pl.ds(v * 16, 16)] = zv
        return carry

    lax.fori_loop(0, K, zero_row, 0)
    for i in range(RPT // K):
        pltpu.sync_copy(msgs_a, acc.at[pl.ds(s * RPT + i * K, K)])
    plsc.subcore_barrier()

    def fetch(t, idx, vals_v):
        pltpu.sync_copy(pk_hbm.at[t], idx)
        pltpu.sync_copy(vals_hbm.at[t], vals_v)

    def issue_gather(idx, msgs, sem):
        pltpu.async_copy(support_hbm.at[idx.at[0]], msgs, sem)

    def wait_gather(idx, msgs, sem):
        pltpu.make_async_copy(support_hbm.at[idx.at[0]], msgs, sem).wait()

    def process(idx, vals_v, msgs):
        def group_body(t, carry):
            base = t * 16
            vv = vals_v[pl.ds(base, 16)]
            for l in range(16):
                val = vv[l]
                for v in range(F // 16):
                    sl = pl.ds(v * 16, 16)
                    msgs[base + l, sl] = msgs[base + l, sl] * val
            return carry

        lax.fori_loop(0, K // 16, group_body, 0)
        pltpu.sync_copy(msgs, acc.at[idx.at[1]], add=True)

    fetch(tbase, idx_a, vals_va)
    issue_gather(idx_a, msgs_a, sem_a)

    def loop_body(j, carry):
        g0 = 2 * j
        fetch(tbase + g0 + 1, idx_b, vals_vb)
        issue_gather(idx_b, msgs_b, sem_b)
        wait_gather(idx_a, msgs_a, sem_a)
        process(idx_a, vals_va, msgs_a)
        fetch(tbase + g0 + 2, idx_a, vals_va)
        issue_gather(idx_a, msgs_a, sem_a)
        wait_gather(idx_b, msgs_b, sem_b)
        process(idx_b, vals_vb, msgs_b)
        return carry

    lax.fori_loop(0, (CHUNKS - 1) // 2, loop_body, 0)
    wait_gather(idx_a, msgs_a, sem_a)
    process(idx_a, vals_va, msgs_a)
    plsc.subcore_barrier()

    for i in range(RPT // K):
        base = s * RPT + i * K
        pltpu.sync_copy(acc.at[pl.ds(base, K)], msgs_a)
        pltpu.sync_copy(msgs_a, out_hbm.at[c, pl.ds(base, K)])


def _spmm(support, rows, cols, vals):
    mesh = plsc.VectorSubcoreMesh(core_axis_name="c", subcore_axis_name="s",
                                  num_cores=NC, num_subcores=NS)
    packed = jnp.stack([cols.reshape(T, K), rows.reshape(T, K)], axis=1)
    f = pl.kernel(
        _spmm_body,
        out_type=jax.ShapeDtypeStruct((NC, NP, F), jnp.float32),
        mesh=mesh,
        scratch_types=[
            pltpu.VMEM((2, K), jnp.int32),
            pltpu.VMEM((2, K), jnp.int32),
            pltpu.VMEM((K,), jnp.float32),
            pltpu.VMEM((K,), jnp.float32),
            pltpu.VMEM((K, F), jnp.float32),
            pltpu.VMEM((K, F), jnp.float32),
            pltpu.VMEM_SHARED((NP, F), jnp.float32),
            pltpu.SemaphoreType.DMA,
            pltpu.SemaphoreType.DMA,
        ],
    )
    return f(support, packed, vals.reshape(T, K))


@jax.jit
def kernel(adj_indices, adj_values, x, W, b):
    x = x.astype(jnp.float32)
    support = _matmul(x, W)
    partials = _spmm(support, adj_indices[0], adj_indices[1], adj_values)
    return _final_add(partials, b)


# speedup vs baseline: 7.8813x; 1.1491x over previous
import jax
import jax.numpy as jnp
from jax import lax
from jax.experimental import pallas as pl
from jax.experimental.pallas import tpu as pltpu
from jax.experimental.pallas import tpu_sc as plsc

N = 10000
E = 320000
F = 128

NC = 2
NS = 16
NW = NC * NS
EPT = E // NW
K = 80
CHUNKS = EPT // K
T = E // K
NP = 10240
RPT = NP // NS


def _mm_body(x_ref, w_ref, o_ref):
    o_ref[...] = jnp.dot(x_ref[...], w_ref[...],
                         preferred_element_type=jnp.float32)


def _matmul(x, W):
    return pl.pallas_call(
        _mm_body,
        grid=(10,),
        in_specs=[
            pl.BlockSpec((N // 10, F), lambda i: (i, 0)),
            pl.BlockSpec((F, F), lambda i: (0, 0)),
        ],
        out_specs=pl.BlockSpec((N // 10, F), lambda i: (i, 0)),
        out_shape=jax.ShapeDtypeStruct((N, F), jnp.float32),
    )(x, W)


def _add_body(p_ref, b_ref, o_ref):
    o_ref[...] = p_ref[0] + p_ref[1] + b_ref[...]


def _final_add(partials, b):
    return pl.pallas_call(
        _add_body,
        grid=(10,),
        in_specs=[
            pl.BlockSpec((2, N // 10, F), lambda i: (0, i, 0)),
            pl.BlockSpec((1, F), lambda i: (0, 0)),
        ],
        out_specs=pl.BlockSpec((N // 10, F), lambda i: (i, 0)),
        out_shape=jax.ShapeDtypeStruct((N, F), jnp.float32),
    )(partials, b.reshape(1, F))


def _spmm_body(support_hbm, pk_hbm, vals_hbm, out_hbm,
               idx_a, idx_b, vals_va, vals_vb, msgs_a, msgs_b, acc,
               sem_a, sem_b, sem_sa, sem_sb):
    c = lax.axis_index("c")
    s = lax.axis_index("s")
    tbase = (c * NS + s) * CHUNKS

    zv = jnp.zeros((16,), jnp.float32)

    def zero_row(j, carry):
        for v in range(F // 16):
            msgs_a[j, pl.ds(v * 16, 16)] = zv
        return carry

    lax.fori_loop(0, K, zero_row, 0)
    for i in range(RPT // K):
        pltpu.sync_copy(msgs_a, acc.at[pl.ds(s * RPT + i * K, K)])
    plsc.subcore_barrier()

    def afetch(g, idx, vals_v, sem):
        t = jnp.minimum(tbase + g, T - 1)
        pltpu.async_copy(pk_hbm.at[t], idx, sem)
        pltpu.async_copy(vals_hbm.at[t], vals_v, sem)

    def wait_fetch(g, idx, vals_v, sem):
        t = jnp.minimum(tbase + g, T - 1)
        pltpu.make_async_copy(pk_hbm.at[t], idx, sem).wait()
        pltpu.make_async_copy(vals_hbm.at[t], vals_v, sem).wait()

    def issue_gather(idx, msgs, sem):
        pltpu.async_copy(support_hbm.at[idx.at[0]], msgs, sem)

    def wait_gather(idx, msgs, sem):
        pltpu.make_async_copy(support_hbm.at[idx.at[0]], msgs, sem).wait()

    def scale(vals_v, msgs):
        def group_body(t, carry):
            base = t * 16
            vv = vals_v[pl.ds(base, 16)]
            for l in range(16):
                val = vv[l]
                for v in range(F // 16):
                    sl = pl.ds(v * 16, 16)
                    msgs[base + l, sl] = msgs[base + l, sl] * val
            return carry

        lax.fori_loop(0, K // 16, group_body, 0)

    def issue_scatter(idx, msgs, sem):
        pltpu.async_copy(msgs, acc.at[idx.at[1]], sem, add=True)

    def wait_scatter(idx, msgs, sem):
        pltpu.make_async_copy(msgs, acc.at[idx.at[1]], sem).wait()

    bufs = ((idx_a, vals_va, msgs_a, sem_a, sem_sa),
            (idx_b, vals_vb, msgs_b, sem_b, sem_sb))

    def half(cur, oth, g):
        ci, cv, cm, cg, cs = cur
        oi, ov, om, og, _ = oth
        wait_gather(ci, cm, cg)
        scale(cv, cm)
        issue_scatter(ci, cm, cs)
        wait_fetch(g + 1, oi, ov, og)
        issue_gather(oi, om, og)
        wait_scatter(ci, cm, cs)
        afetch(g + 2, ci, cv, cg)

    afetch(0, idx_a, vals_va, sem_a)
    wait_fetch(0, idx_a, vals_va, sem_a)
    issue_gather(idx_a, msgs_a, sem_a)
    afetch(1, idx_b, vals_vb, sem_b)

    def loop_body(j, carry):
        half(bufs[0], bufs[1], 2 * j)
        half(bufs[1], bufs[0], 2 * j + 1)
        return carry

    lax.fori_loop(0, (CHUNKS - 1) // 2, loop_body, 0)
    wait_gather(idx_a, msgs_a, sem_a)
    scale(vals_va, msgs_a)
    issue_scatter(idx_a, msgs_a, sem_sa)
    wait_scatter(idx_a, msgs_a, sem_sa)
    wait_fetch(CHUNKS, idx_b, vals_vb, sem_b)
    plsc.subcore_barrier()

    for i in range(RPT // K):
        base = s * RPT + i * K
        pltpu.sync_copy(acc.at[pl.ds(base, K)], msgs_a)
        pltpu.sync_copy(msgs_a, out_hbm.at[c, pl.ds(base, K)])


def _spmm(support, rows, cols, vals):
    mesh = plsc.VectorSubcoreMesh(core_axis_name="c", subcore_axis_name="s",
                                  num_cores=NC, num_subcores=NS)
    packed = jnp.stack([cols.reshape(T, K), rows.reshape(T, K)], axis=1)
    f = pl.kernel(
        _spmm_body,
        out_type=jax.ShapeDtypeStruct((NC, NP, F), jnp.float32),
        mesh=mesh,
        scratch_types=[
            pltpu.VMEM((2, K), jnp.int32),
            pltpu.VMEM((2, K), jnp.int32),
            pltpu.VMEM((K,), jnp.float32),
            pltpu.VMEM((K,), jnp.float32),
            pltpu.VMEM((K, F), jnp.float32),
            pltpu.VMEM((K, F), jnp.float32),
            pltpu.VMEM_SHARED((NP, F), jnp.float32),
            pltpu.SemaphoreType.DMA,
            pltpu.SemaphoreType.DMA,
            pltpu.SemaphoreType.DMA,
            pltpu.SemaphoreType.DMA,
        ],
    )
    return f(support, packed, vals.reshape(T, K))


@jax.jit
def kernel(adj_indices, adj_values, x, W, b):
    x = x.astype(jnp.float32)
    support = _matmul(x, W)
    partials = _spmm(support, adj_indices[0], adj_indices[1], adj_values)
    return _final_add(partials, b)
